# baseline (speedup 1.0000x reference)
import jax
import jax.numpy as jnp
from jax import lax
from jax.experimental import pallas as pl
from jax.experimental.pallas import tpu as pltpu
from jax.experimental.pallas import tpu_sc as plsc

N = 10000
E = 160000
EMB = 300
D = 320
H = 640
DH = D // 2
VOC = 256
NLAYER = 5
FEAT = 512

NC, NS = 2, 16
B = 80
NBATCH = 125
NBUF = 2
NAGG = 10008
RPT = 632
RPT_LAST = NAGG - 15 * RPT

RBLK = 1000

_VOCABS = [119, 8, 12, 15, 10, 6, 7, 3, 3]
_OFFS = [0]
for _v in _VOCABS[:-1]:
    _OFFS.append(_OFFS[-1] + _v)


def _pad2(a, rows, cols):
    return jnp.pad(a, ((0, rows - a.shape[0]), (0, cols - a.shape[1])))


def _padrow(a, cols):
    return jnp.pad(a.reshape(1, -1), ((0, 0), (0, cols - a.shape[0])))


def _segsum_body(h2, sd3, out, sd_v, idx_v, dst_v, rows_v, aggr,
                 gsem, ssem, dsem):
    c = lax.axis_index("c")
    s = lax.axis_index("s")

    zero16 = jnp.zeros((16,), jnp.float32)

    def _zrow(r, carry):
        for j in range(DH // 16):
            rows_v[0, r, pl.ds(j * 16, 16)] = zero16
        return carry

    lax.fori_loop(0, B, _zrow, 0)
    zbase = s * RPT
    for k in range(6):
        pltpu.sync_copy(rows_v.at[0], aggr.at[pl.ds(zbase + k * B, B)])

    @pl.when(s < NS - 1)
    def _():
        pltpu.sync_copy(rows_v.at[0], aggr.at[pl.ds(zbase + 6 * B, B)])
        pltpu.sync_copy(rows_v.at[0, pl.ds(0, RPT - 7 * B)],
                        aggr.at[pl.ds(zbase + 7 * B, RPT - 7 * B)])

    @pl.when(s == NS - 1)
    def _():
        pltpu.sync_copy(rows_v.at[0, pl.ds(0, RPT_LAST - 6 * B)],
                        aggr.at[pl.ds(zbase + 6 * B, RPT_LAST - 6 * B)])

    plsc.subcore_barrier()

    def _sdstart(j, slot):
        pltpu.make_async_copy(sd3.at[s, j], sd_v.at[slot],
                              dsem.at[slot]).start()

    def _sdwait(j, slot):
        pltpu.make_async_copy(sd3.at[s, j], sd_v.at[slot],
                              dsem.at[slot]).wait()

    def _unpack(slot):
        for k in range(B // 16):
            v = sd_v[slot, pl.ds(k * 16, 16)]
            idx_v[slot, pl.ds(k * 16, 16)] = (v >> 14) * 2 + c
            dst_v[slot, pl.ds(k * 16, 16)] = v & 16383

    def _gstart(slot):
        pltpu.make_async_copy(h2.at[idx_v.at[slot]], rows_v.at[slot],
                              gsem.at[slot]).start()

    def _gwait(slot):
        pltpu.make_async_copy(h2.at[idx_v.at[slot]], rows_v.at[slot],
                              gsem.at[slot]).wait()

    def _sstart(slot):
        pltpu.make_async_copy(rows_v.at[slot], aggr.at[dst_v.at[slot]],
                              ssem.at[slot]).start(add=True)

    def _swait(slot):
        pltpu.make_async_copy(rows_v.at[slot], aggr.at[dst_v.at[slot]],
                              ssem.at[slot]).wait()

    for b in range(NBUF):
        _sdstart(b, b)
        _sdwait(b, b)
        _unpack(b)
        _gstart(b)

    def _pair(p, carry):
        for b in range(NBUF):
            j = p * NBUF + b

            @pl.when(j < NBATCH)
            def _():
                jn = jnp.minimum(j + NBUF, NBATCH - 1)
                _gwait(b)
                _sstart(b)

                @pl.when(j + NBUF < NBATCH)
                def _():
                    _sdstart(jn, b)
                _swait(b)

                @pl.when(j + NBUF < NBATCH)
                def _():
                    _sdwait(jn, b)
                    _unpack(b)
                    _gstart(b)
        return carry

    lax.fori_loop(0, (NBATCH + NBUF - 1) // NBUF, _pair, 0)
    plsc.subcore_barrier()

    obase = s * RPT

    @pl.when(s < NS - 1)
    def _():
        pltpu.sync_copy(aggr.at[pl.ds(obase, RPT)],
                        out.at[c, pl.ds(obase, RPT)])

    @pl.when(s == NS - 1)
    def _():
        pltpu.sync_copy(aggr.at[pl.ds(obase, RPT_LAST)],
                        out.at[c, pl.ds(obase, RPT_LAST)])


_segsum_cache = []


def _segsum(h2, sd3):
    if not _segsum_cache:
        _segsum_cache.append(pl.kernel(
            _segsum_body,
            out_type=jax.ShapeDtypeStruct((NC, NAGG, DH), jnp.float32),
            mesh=plsc.VectorSubcoreMesh(core_axis_name="c",
                                        subcore_axis_name="s"),
            scratch_types=[
                pltpu.VMEM((NBUF, B), jnp.int32),
                pltpu.VMEM((NBUF, B), jnp.int32),
                pltpu.VMEM((NBUF, B), jnp.int32),
                pltpu.VMEM((NBUF, B, DH), jnp.float32),
                pltpu.VMEM_SHARED((NAGG, DH), jnp.float32),
                pltpu.SemaphoreType.DMA((NBUF,)),
                pltpu.SemaphoreType.DMA((NBUF,)),
                pltpu.SemaphoreType.DMA((NBUF,)),
            ],
            compiler_params=pltpu.CompilerParams(use_tc_tiling_on_sc=False),
        ))
    return _segsum_cache[0](h2, sd3)


def _emb_body(x_ref, tab_ref, out_ref):
    xb = x_ref[...]
    iota = lax.broadcasted_iota(jnp.int32, (1, VOC), 1)
    mh = (xb[:, 0:1] == iota).astype(jnp.float32)
    for i in range(1, 9):
        mh = mh + (xb[:, i:i + 1] == iota).astype(jnp.float32)
    out_ref[...] = jnp.dot(mh, tab_ref[...], preferred_element_type=jnp.float32)


def _emb(xoff, tabp):
    return pl.pallas_call(
        _emb_body,
        grid=(N // RBLK,),
        in_specs=[
            pl.BlockSpec((RBLK, 9), lambda i: (i, 0)),
            pl.BlockSpec((VOC, D), lambda i: (0, 0)),
        ],
        out_specs=pl.BlockSpec((RBLK, D), lambda i: (i, 0)),
        out_shape=jax.ShapeDtypeStruct((N, D), jnp.float32),
    )(xoff, tabp)


def _dense_body(h_ref, a_ref, er_ref, w1_ref, b1_ref, w2_ref, b2_ref,
                hout_ref, s_ref, q_ref):
    i = pl.program_id(0)
    a = a_ref[...]
    aggr = jnp.concatenate([a[0], a[1]], axis=1)
    z = er_ref[...] * h_ref[...] + aggr
    t = jnp.maximum(
        jnp.dot(z, w1_ref[...], preferred_element_type=jnp.float32)
        + b1_ref[...], 0.0)
    hu = (jnp.dot(t, w2_ref[...], preferred_element_type=jnp.float32)
          + b2_ref[...])
    hout_ref[...] = hu
    ps = jnp.sum(hu, axis=0, keepdims=True)
    pq = jnp.sum(hu * hu, axis=0, keepdims=True)

    @pl.when(i == 0)
    def _():
        s_ref[...] = ps
        q_ref[...] = pq

    @pl.when(i != 0)
    def _():
        s_ref[...] = s_ref[...] + ps
        q_ref[...] = q_ref[...] + pq


def _dense(h, aggr2, er, w1p, b1p, w2p, b2p):
    return pl.pallas_call(
        _dense_body,
        grid=(N // RBLK,),
        in_specs=[
            pl.BlockSpec((RBLK, D), lambda i: (i, 0)),
            pl.BlockSpec((NC, RBLK, DH), lambda i: (0, i, 0)),
            pl.BlockSpec((1, D), lambda i: (0, 0)),
            pl.BlockSpec((D, H), lambda i: (0, 0)),
            pl.BlockSpec((1, H), lambda i: (0, 0)),
            pl.BlockSpec((H, D), lambda i: (0, 0)),
            pl.BlockSpec((1, D), lambda i: (0, 0)),
        ],
        out_specs=[
            pl.BlockSpec((RBLK, D), lambda i: (i, 0)),
            pl.BlockSpec((1, D), lambda i: (0, 0)),
            pl.BlockSpec((1, D), lambda i: (0, 0)),
        ],
        out_shape=[
            jax.ShapeDtypeStruct((N, D), jnp.float32),
            jax.ShapeDtypeStruct((1, D), jnp.float32),
            jax.ShapeDtypeStruct((1, D), jnp.float32),
        ],
    )(h, aggr2, er, w1p, b1p, w2p, b2p)


def _bn_relu_body(h_ref, s_ref, q_ref, g_ref, b_ref, out_ref):
    mean = s_ref[...] * (1.0 / N)
    var = q_ref[...] * (1.0 / N) - mean * mean
    inv = g_ref[...] * lax.rsqrt(var + 1e-5)
    y = (h_ref[...] - mean) * inv + b_ref[...]
    out_ref[...] = jnp.maximum(y, 0.0)


def _bn_relu(hu, s, q, g, b):
    return pl.pallas_call(
        _bn_relu_body,
        grid=(N // RBLK,),
        in_specs=[
            pl.BlockSpec((RBLK, D), lambda i: (i, 0)),
            pl.BlockSpec((1, D), lambda i: (0, 0)),
            pl.BlockSpec((1, D), lambda i: (0, 0)),
            pl.BlockSpec((1, D), lambda i: (0, 0)),
            pl.BlockSpec((1, D), lambda i: (0, 0)),
        ],
        out_specs=pl.BlockSpec((RBLK, D), lambda i: (i, 0)),
        out_shape=jax.ShapeDtypeStruct((N, D), jnp.float32),
    )(hu, s, q, g, b)


def _dense_last_body(h_ref, a_ref, er_ref, w1_ref, b1_ref, w2_ref, b2_ref,
                     s_ref, q_ref):
    i = pl.program_id(0)
    a = a_ref[...]
    aggr = jnp.concatenate([a[0], a[1]], axis=1)
    z = er_ref[...] * h_ref[...] + aggr
    t = jnp.maximum(
        jnp.dot(z, w1_ref[...], preferred_element_type=jnp.float32)
        + b1_ref[...], 0.0)
    hu = (jnp.dot(t, w2_ref[...], preferred_element_type=jnp.float32)
          + b2_ref[...])
    ps = jnp.sum(hu, axis=0, keepdims=True)
    pq = jnp.sum(hu * hu, axis=0, keepdims=True)

    @pl.when(i == 0)
    def _():
        s_ref[...] = ps
        q_ref[...] = pq

    @pl.when(i != 0)
    def _():
        s_ref[...] = s_ref[...] + ps
        q_ref[...] = q_ref[...] + pq


def _dense_last(h, aggr2, er, w1p, b1p, w2p, b2p):
    return pl.pallas_call(
        _dense_last_body,
        grid=(N // RBLK,),
        in_specs=[
            pl.BlockSpec((RBLK, D), lambda i: (i, 0)),
            pl.BlockSpec((NC, RBLK, DH), lambda i: (0, i, 0)),
            pl.BlockSpec((1, D), lambda i: (0, 0)),
            pl.BlockSpec((D, H), lambda i: (0, 0)),
            pl.BlockSpec((1, H), lambda i: (0, 0)),
            pl.BlockSpec((H, D), lambda i: (0, 0)),
            pl.BlockSpec((1, D), lambda i: (0, 0)),
        ],
        out_specs=[
            pl.BlockSpec((1, D), lambda i: (0, 0)),
            pl.BlockSpec((1, D), lambda i: (0, 0)),
        ],
        out_shape=[
            jax.ShapeDtypeStruct((1, D), jnp.float32),
            jax.ShapeDtypeStruct((1, D), jnp.float32),
        ],
    )(h, aggr2, er, w1p, b1p, w2p, b2p)


def _softplus(x):
    return jnp.maximum(x, 0.0) + jnp.log1p(jnp.exp(-jnp.abs(x)))


def _head_body(s_ref, q_ref, g4_ref, b4_ref, fw_ref, fb_ref, w0_ref, b0_ref,
               w1_ref, b1_ref, w2_ref, b2_ref, out_ref):
    mean = s_ref[...] * (1.0 / N)
    var = q_ref[...] * (1.0 / N) - mean * mean
    inv = g4_ref[...] * lax.rsqrt(var + 1e-5)
    g = (s_ref[...] * (1.0 / N) - mean) * inv + b4_ref[...]
    f = (jnp.dot(g, fw_ref[...], preferred_element_type=jnp.float32)
         + fb_ref[...])
    f = _softplus(jnp.dot(f, w0_ref[...], preferred_element_type=jnp.float32)
                  + b0_ref[...])
    f = _softplus(jnp.dot(f, w1_ref[...], preferred_element_type=jnp.float32)
                  + b1_ref[...])
    out_ref[...] = (jnp.dot(f, w2_ref[...], preferred_element_type=jnp.float32)
                    + b2_ref[...])


def _head(s, q, g4, b4, fw, fb, w0, b0, w1, b1, w2, b2):
    return pl.pallas_call(
        _head_body,
        out_shape=jax.ShapeDtypeStruct((1, 128), jnp.float32),
    )(s, q, g4, b4, fw, fb, w0, b0, w1, b1, w2, b2)


def kernel(x, edge_index, params):
    offs = jnp.asarray(_OFFS, jnp.int32)
    xoff = x.astype(jnp.int32) + offs[None, :]
    tabp = _pad2(jnp.concatenate(params["emb"], axis=0), VOC, D)

    src = edge_index[0].astype(jnp.int32)
    dst = edge_index[1].astype(jnp.int32)
    sd = (src << 14) | dst
    sd3 = sd.reshape(NS, NBATCH, B)

    h = _emb(xoff, tabp)
    for l in range(NLAYER):
        p = params["gnn"][l]
        aggr2 = _segsum(h.reshape(2 * N, DH), sd3)
        er = jnp.broadcast_to(1.0 + p["eps"], (1, D))
        w1p = _pad2(p["W1"], D, H)
        b1p = _padrow(p["b1"], H)
        w2p = _pad2(p["W2"], H, D)
        b2p = _padrow(p["b2"], D)
        gp = _padrow(p["bn_g"], D)
        bp = _padrow(p["bn_b"], D)
        if l < NLAYER - 1:
            hu, s, q = _dense(h, aggr2, er, w1p, b1p, w2p, b2p)
            h = _bn_relu(hu, s, q, gp, bp)
        else:
            s4, q4 = _dense_last(h, aggr2, er, w1p, b1p, w2p, b2p)
            g4, b4 = gp, bp

    fw = _pad2(params["feat_W"], D, FEAT)
    fb = _padrow(params["feat_b"], FEAT)
    w0 = params["p0W"]
    b0 = _padrow(params["p0b"], FEAT // 2)
    w1 = params["p1W"]
    b1 = _padrow(params["p1b"], FEAT // 2)
    w2 = _pad2(params["p2W"], FEAT // 2, 128)
    b2 = _padrow(params["p2b"], 128)
    out = _head(s4, q4, g4, b4, fw, fb, w0, b0, w1, b1, w2, b2)
    return out[:, :2]


# speedup vs baseline: 1.0096x; 1.0096x over previous
import jax
import jax.numpy as jnp
from jax import lax
from jax.experimental import pallas as pl
from jax.experimental.pallas import tpu as pltpu
from jax.experimental.pallas import tpu_sc as plsc

N = 10000
E = 160000
EMB = 300
D = 320
H = 640
DH = D // 2
VOC = 256
NLAYER = 5
FEAT = 512

NC, NS = 2, 16
B = 80
NBATCH = 125
NBUF = 2
NAGG = 10008
RPT = 632
RPT_LAST = NAGG - 15 * RPT

RBLK = 1000

_VOCABS = [119, 8, 12, 15, 10, 6, 7, 3, 3]
_OFFS = [0]
for _v in _VOCABS[:-1]:
    _OFFS.append(_OFFS[-1] + _v)


def _pad2(a, rows, cols):
    return jnp.pad(a, ((0, rows - a.shape[0]), (0, cols - a.shape[1])))


def _padrow(a, cols):
    return jnp.pad(a.reshape(1, -1), ((0, 0), (0, cols - a.shape[0])))


def _segsum_body(h2, sd3, out, sd_v, idx_v, dst_v, rows_v, aggr,
                 gsem, ssem, dsem):
    c = lax.axis_index("c")
    s = lax.axis_index("s")

    zero16 = jnp.zeros((16,), jnp.float32)

    def _zrow(r, carry):
        for j in range(DH // 16):
            rows_v[0, r, pl.ds(j * 16, 16)] = zero16
        return carry

    lax.fori_loop(0, B, _zrow, 0)
    zbase = s * RPT
    for k in range(6):
        pltpu.sync_copy(rows_v.at[0], aggr.at[pl.ds(zbase + k * B, B)])

    @pl.when(s < NS - 1)
    def _():
        pltpu.sync_copy(rows_v.at[0], aggr.at[pl.ds(zbase + 6 * B, B)])
        pltpu.sync_copy(rows_v.at[0, pl.ds(0, RPT - 7 * B)],
                        aggr.at[pl.ds(zbase + 7 * B, RPT - 7 * B)])

    @pl.when(s == NS - 1)
    def _():
        pltpu.sync_copy(rows_v.at[0, pl.ds(0, RPT_LAST - 6 * B)],
                        aggr.at[pl.ds(zbase + 6 * B, RPT_LAST - 6 * B)])

    plsc.subcore_barrier()

    def _sdstart(j, slot):
        pltpu.make_async_copy(sd3.at[s, j], sd_v.at[slot],
                              dsem.at[slot]).start()

    def _sdwait(j, slot):
        pltpu.make_async_copy(sd3.at[s, j], sd_v.at[slot],
                              dsem.at[slot]).wait()

    def _unpack(slot):
        for k in range(B // 16):
            v = sd_v[slot, pl.ds(k * 16, 16)]
            idx_v[slot, pl.ds(k * 16, 16)] = (v >> 14) * 2 + c
            dst_v[slot, pl.ds(k * 16, 16)] = v & 16383

    def _gstart(slot):
        pltpu.make_async_copy(h2.at[idx_v.at[slot]], rows_v.at[slot],
                              gsem.at[slot]).start()

    def _gwait(slot):
        pltpu.make_async_copy(h2.at[idx_v.at[slot]], rows_v.at[slot],
                              gsem.at[slot]).wait()

    def _sstart(slot):
        pltpu.make_async_copy(rows_v.at[slot], aggr.at[dst_v.at[slot]],
                              ssem.at[slot]).start(add=True)

    def _swait(slot):
        pltpu.make_async_copy(rows_v.at[slot], aggr.at[dst_v.at[slot]],
                              ssem.at[slot]).wait()

    for b in range(NBUF):
        _sdstart(b, b)
        _sdwait(b, b)
        _unpack(b)
        _gstart(b)

    def _pair(p, carry):
        for b in range(NBUF):
            j = p * NBUF + b

            @pl.when(j < NBATCH)
            def _():
                jn = jnp.minimum(j + NBUF, NBATCH - 1)
                _gwait(b)
                _sstart(b)

                @pl.when(j + NBUF < NBATCH)
                def _():
                    _sdstart(jn, b)
                _swait(b)

                @pl.when(j + NBUF < NBATCH)
                def _():
                    _sdwait(jn, b)
                    _unpack(b)
                    _gstart(b)
        return carry

    lax.fori_loop(0, (NBATCH + NBUF - 1) // NBUF, _pair, 0)
    plsc.subcore_barrier()

    obase = s * RPT

    def _wb(k, slot):
        pltpu.sync_copy(aggr.at[pl.ds(obase + k * B, B)], rows_v.at[slot])
        pltpu.make_async_copy(rows_v.at[slot],
                              out.at[c, pl.ds(obase + k * B, B)],
                              gsem.at[slot]).start()

    def _wbwait(k, slot):
        pltpu.make_async_copy(rows_v.at[slot],
                              out.at[c, pl.ds(obase + k * B, B)],
                              gsem.at[slot]).wait()

    for k in range(6):
        if k >= 2:
            _wbwait(k - 2, k & 1)
        _wb(k, k & 1)

    @pl.when(s < NS - 1)
    def _():
        _wbwait(4, 0)
        _wb(6, 0)
        _wbwait(5, 1)
        n_tail = RPT - 7 * B
        pltpu.sync_copy(aggr.at[pl.ds(obase + 7 * B, n_tail)],
                        rows_v.at[1, pl.ds(0, n_tail)])
        pltpu.sync_copy(rows_v.at[1, pl.ds(0, n_tail)],
                        out.at[c, pl.ds(obase + 7 * B, n_tail)])
        _wbwait(6, 0)

    @pl.when(s == NS - 1)
    def _():
        _wbwait(4, 0)
        _wbwait(5, 1)
        n_tail = RPT_LAST - 6 * B
        pltpu.sync_copy(aggr.at[pl.ds(obase + 6 * B, n_tail)],
                        rows_v.at[0, pl.ds(0, n_tail)])
        pltpu.sync_copy(rows_v.at[0, pl.ds(0, n_tail)],
                        out.at[c, pl.ds(obase + 6 * B, n_tail)])


_segsum_cache = []


def _segsum(h2, sd3):
    if not _segsum_cache:
        _segsum_cache.append(pl.kernel(
            _segsum_body,
            out_type=jax.ShapeDtypeStruct((NC, NAGG, DH), jnp.float32),
            mesh=plsc.VectorSubcoreMesh(core_axis_name="c",
                                        subcore_axis_name="s"),
            scratch_types=[
                pltpu.VMEM((NBUF, B), jnp.int32),
                pltpu.VMEM((NBUF, B), jnp.int32),
                pltpu.VMEM((NBUF, B), jnp.int32),
                pltpu.VMEM((NBUF, B, DH), jnp.float32),
                pltpu.VMEM_SHARED((NAGG, DH), jnp.float32),
                pltpu.SemaphoreType.DMA((NBUF,)),
                pltpu.SemaphoreType.DMA((NBUF,)),
                pltpu.SemaphoreType.DMA((NBUF,)),
            ],
            compiler_params=pltpu.CompilerParams(use_tc_tiling_on_sc=False),
        ))
    return _segsum_cache[0](h2, sd3)


def _emb_body(x_ref, tab_ref, out_ref):
    xb = x_ref[...]
    iota = lax.broadcasted_iota(jnp.int32, (1, VOC), 1)
    mh = (xb[:, 0:1] == iota).astype(jnp.float32)
    for i in range(1, 9):
        mh = mh + (xb[:, i:i + 1] == iota).astype(jnp.float32)
    out_ref[...] = jnp.dot(mh, tab_ref[...], preferred_element_type=jnp.float32)


def _emb(xoff, tabp):
    return pl.pallas_call(
        _emb_body,
        grid=(N // RBLK,),
        in_specs=[
            pl.BlockSpec((RBLK, 9), lambda i: (i, 0)),
            pl.BlockSpec((VOC, D), lambda i: (0, 0)),
        ],
        out_specs=pl.BlockSpec((RBLK, D), lambda i: (i, 0)),
        out_shape=jax.ShapeDtypeStruct((N, D), jnp.float32),
    )(xoff, tabp)


def _dense_body(h_ref, a_ref, er_ref, w1_ref, b1_ref, w2_ref, b2_ref,
                hout_ref, s_ref, q_ref):
    i = pl.program_id(0)
    a = a_ref[...]
    aggr = jnp.concatenate([a[0], a[1]], axis=1)
    z = er_ref[...] * h_ref[...] + aggr
    t = jnp.maximum(
        jnp.dot(z, w1_ref[...], preferred_element_type=jnp.float32)
        + b1_ref[...], 0.0)
    hu = (jnp.dot(t, w2_ref[...], preferred_element_type=jnp.float32)
          + b2_ref[...])
    hout_ref[...] = hu
    ps = jnp.sum(hu, axis=0, keepdims=True)
    pq = jnp.sum(hu * hu, axis=0, keepdims=True)

    @pl.when(i == 0)
    def _():
        s_ref[...] = ps
        q_ref[...] = pq

    @pl.when(i != 0)
    def _():
        s_ref[...] = s_ref[...] + ps
        q_ref[...] = q_ref[...] + pq


def _dense(h, aggr2, er, w1p, b1p, w2p, b2p):
    return pl.pallas_call(
        _dense_body,
        grid=(N // RBLK,),
        in_specs=[
            pl.BlockSpec((RBLK, D), lambda i: (i, 0)),
            pl.BlockSpec((NC, RBLK, DH), lambda i: (0, i, 0)),
            pl.BlockSpec((1, D), lambda i: (0, 0)),
            pl.BlockSpec((D, H), lambda i: (0, 0)),
            pl.BlockSpec((1, H), lambda i: (0, 0)),
            pl.BlockSpec((H, D), lambda i: (0, 0)),
            pl.BlockSpec((1, D), lambda i: (0, 0)),
        ],
        out_specs=[
            pl.BlockSpec((RBLK, D), lambda i: (i, 0)),
            pl.BlockSpec((1, D), lambda i: (0, 0)),
            pl.BlockSpec((1, D), lambda i: (0, 0)),
        ],
        out_shape=[
            jax.ShapeDtypeStruct((N, D), jnp.float32),
            jax.ShapeDtypeStruct((1, D), jnp.float32),
            jax.ShapeDtypeStruct((1, D), jnp.float32),
        ],
    )(h, aggr2, er, w1p, b1p, w2p, b2p)


def _bn_relu_body(h_ref, s_ref, q_ref, g_ref, b_ref, out_ref):
    mean = s_ref[...] * (1.0 / N)
    var = q_ref[...] * (1.0 / N) - mean * mean
    inv = g_ref[...] * lax.rsqrt(var + 1e-5)
    y = (h_ref[...] - mean) * inv + b_ref[...]
    out_ref[...] = jnp.maximum(y, 0.0)


def _bn_relu(hu, s, q, g, b):
    return pl.pallas_call(
        _bn_relu_body,
        grid=(N // RBLK,),
        in_specs=[
            pl.BlockSpec((RBLK, D), lambda i: (i, 0)),
            pl.BlockSpec((1, D), lambda i: (0, 0)),
            pl.BlockSpec((1, D), lambda i: (0, 0)),
            pl.BlockSpec((1, D), lambda i: (0, 0)),
            pl.BlockSpec((1, D), lambda i: (0, 0)),
        ],
        out_specs=pl.BlockSpec((RBLK, D), lambda i: (i, 0)),
        out_shape=jax.ShapeDtypeStruct((N, D), jnp.float32),
    )(hu, s, q, g, b)


def _dense_last_body(h_ref, a_ref, er_ref, w1_ref, b1_ref, w2_ref, b2_ref,
                     s_ref, q_ref):
    i = pl.program_id(0)
    a = a_ref[...]
    aggr = jnp.concatenate([a[0], a[1]], axis=1)
    z = er_ref[...] * h_ref[...] + aggr
    t = jnp.maximum(
        jnp.dot(z, w1_ref[...], preferred_element_type=jnp.float32)
        + b1_ref[...], 0.0)
    hu = (jnp.dot(t, w2_ref[...], preferred_element_type=jnp.float32)
          + b2_ref[...])
    ps = jnp.sum(hu, axis=0, keepdims=True)
    pq = jnp.sum(hu * hu, axis=0, keepdims=True)

    @pl.when(i == 0)
    def _():
        s_ref[...] = ps
        q_ref[...] = pq

    @pl.when(i != 0)
    def _():
        s_ref[...] = s_ref[...] + ps
        q_ref[...] = q_ref[...] + pq


def _dense_last(h, aggr2, er, w1p, b1p, w2p, b2p):
    return pl.pallas_call(
        _dense_last_body,
        grid=(N // RBLK,),
        in_specs=[
            pl.BlockSpec((RBLK, D), lambda i: (i, 0)),
            pl.BlockSpec((NC, RBLK, DH), lambda i: (0, i, 0)),
            pl.BlockSpec((1, D), lambda i: (0, 0)),
            pl.BlockSpec((D, H), lambda i: (0, 0)),
            pl.BlockSpec((1, H), lambda i: (0, 0)),
            pl.BlockSpec((H, D), lambda i: (0, 0)),
            pl.BlockSpec((1, D), lambda i: (0, 0)),
        ],
        out_specs=[
            pl.BlockSpec((1, D), lambda i: (0, 0)),
            pl.BlockSpec((1, D), lambda i: (0, 0)),
        ],
        out_shape=[
            jax.ShapeDtypeStruct((1, D), jnp.float32),
            jax.ShapeDtypeStruct((1, D), jnp.float32),
        ],
    )(h, aggr2, er, w1p, b1p, w2p, b2p)


def _softplus(x):
    return jnp.maximum(x, 0.0) + jnp.log1p(jnp.exp(-jnp.abs(x)))


def _head_body(s_ref, q_ref, g4_ref, b4_ref, fw_ref, fb_ref, w0_ref, b0_ref,
               w1_ref, b1_ref, w2_ref, b2_ref, out_ref):
    mean = s_ref[...] * (1.0 / N)
    var = q_ref[...] * (1.0 / N) - mean * mean
    inv = g4_ref[...] * lax.rsqrt(var + 1e-5)
    g = (s_ref[...] * (1.0 / N) - mean) * inv + b4_ref[...]
    f = (jnp.dot(g, fw_ref[...], preferred_element_type=jnp.float32)
         + fb_ref[...])
    f = _softplus(jnp.dot(f, w0_ref[...], preferred_element_type=jnp.float32)
                  + b0_ref[...])
    f = _softplus(jnp.dot(f, w1_ref[...], preferred_element_type=jnp.float32)
                  + b1_ref[...])
    out_ref[...] = (jnp.dot(f, w2_ref[...], preferred_element_type=jnp.float32)
                    + b2_ref[...])


def _head(s, q, g4, b4, fw, fb, w0, b0, w1, b1, w2, b2):
    return pl.pallas_call(
        _head_body,
        out_shape=jax.ShapeDtypeStruct((1, 128), jnp.float32),
    )(s, q, g4, b4, fw, fb, w0, b0, w1, b1, w2, b2)


def kernel(x, edge_index, params):
    offs = jnp.asarray(_OFFS, jnp.int32)
    xoff = x.astype(jnp.int32) + offs[None, :]
    tabp = _pad2(jnp.concatenate(params["emb"], axis=0), VOC, D)

    src = edge_index[0].astype(jnp.int32)
    dst = edge_index[1].astype(jnp.int32)
    sd = (src << 14) | dst
    sd3 = sd.reshape(NS, NBATCH, B)

    h = _emb(xoff, tabp)
    for l in range(NLAYER):
        p = params["gnn"][l]
        aggr2 = _segsum(h.reshape(2 * N, DH), sd3)
        er = jnp.broadcast_to(1.0 + p["eps"], (1, D))
        w1p = _pad2(p["W1"], D, H)
        b1p = _padrow(p["b1"], H)
        w2p = _pad2(p["W2"], H, D)
        b2p = _padrow(p["b2"], D)
        gp = _padrow(p["bn_g"], D)
        bp = _padrow(p["bn_b"], D)
        if l < NLAYER - 1:
            hu, s, q = _dense(h, aggr2, er, w1p, b1p, w2p, b2p)
            h = _bn_relu(hu, s, q, gp, bp)
        else:
            s4, q4 = _dense_last(h, aggr2, er, w1p, b1p, w2p, b2p)
            g4, b4 = gp, bp

    fw = _pad2(params["feat_W"], D, FEAT)
    fb = _padrow(params["feat_b"], FEAT)
    w0 = params["p0W"]
    b0 = _padrow(params["p0b"], FEAT // 2)
    w1 = params["p1W"]
    b1 = _padrow(params["p1b"], FEAT // 2)
    w2 = _pad2(params["p2W"], FEAT // 2, 128)
    b2 = _padrow(params["p2b"], 128)
    out = _head(s4, q4, g4, b4, fw, fb, w0, b0, w1, b1, w2, b2)
    return out[:, :2]


# speedup vs baseline: 1.0229x; 1.0131x over previous
import jax
import jax.numpy as jnp
from jax import lax
from jax.experimental import pallas as pl
from jax.experimental.pallas import tpu as pltpu
from jax.experimental.pallas import tpu_sc as plsc

N = 10000
E = 160000
EMB = 300
D = 320
H = 640
DH = D // 2
VOC = 256
NLAYER = 5
FEAT = 512

NC, NS = 2, 16
B = 80
NBATCH = 125
NBUF = 2
NAGG = 10008
RPT = 632
RPT_LAST = NAGG - 15 * RPT

RBLK = 2000

_VOCABS = [119, 8, 12, 15, 10, 6, 7, 3, 3]
_OFFS = [0]
for _v in _VOCABS[:-1]:
    _OFFS.append(_OFFS[-1] + _v)


def _pad2(a, rows, cols):
    return jnp.pad(a, ((0, rows - a.shape[0]), (0, cols - a.shape[1])))


def _padrow(a, cols):
    return jnp.pad(a.reshape(1, -1), ((0, 0), (0, cols - a.shape[0])))


def _segsum_body(h2, sd3, out, sd_v, idx_v, dst_v, rows_v, aggr,
                 gsem, ssem, dsem):
    c = lax.axis_index("c")
    s = lax.axis_index("s")

    zero16 = jnp.zeros((16,), jnp.float32)

    def _zrow(r, carry):
        for j in range(DH // 16):
            rows_v[0, r, pl.ds(j * 16, 16)] = zero16
        return carry

    lax.fori_loop(0, B, _zrow, 0)
    zbase = s * RPT
    for k in range(6):
        pltpu.sync_copy(rows_v.at[0], aggr.at[pl.ds(zbase + k * B, B)])

    @pl.when(s < NS - 1)
    def _():
        pltpu.sync_copy(rows_v.at[0], aggr.at[pl.ds(zbase + 6 * B, B)])
        pltpu.sync_copy(rows_v.at[0, pl.ds(0, RPT - 7 * B)],
                        aggr.at[pl.ds(zbase + 7 * B, RPT - 7 * B)])

    @pl.when(s == NS - 1)
    def _():
        pltpu.sync_copy(rows_v.at[0, pl.ds(0, RPT_LAST - 6 * B)],
                        aggr.at[pl.ds(zbase + 6 * B, RPT_LAST - 6 * B)])

    plsc.subcore_barrier()

    def _sdstart(j, slot):
        pltpu.make_async_copy(sd3.at[s, j], sd_v.at[slot],
                              dsem.at[slot]).start()

    def _sdwait(j, slot):
        pltpu.make_async_copy(sd3.at[s, j], sd_v.at[slot],
                              dsem.at[slot]).wait()

    def _unpack(slot):
        for k in range(B // 16):
            v = sd_v[slot, pl.ds(k * 16, 16)]
            idx_v[slot, pl.ds(k * 16, 16)] = (v >> 14) * 2 + c
            dst_v[slot, pl.ds(k * 16, 16)] = v & 16383

    def _gstart(slot):
        pltpu.make_async_copy(h2.at[idx_v.at[slot]], rows_v.at[slot],
                              gsem.at[slot]).start()

    def _gwait(slot):
        pltpu.make_async_copy(h2.at[idx_v.at[slot]], rows_v.at[slot],
                              gsem.at[slot]).wait()

    def _sstart(slot):
        pltpu.make_async_copy(rows_v.at[slot], aggr.at[dst_v.at[slot]],
                              ssem.at[slot]).start(add=True)

    def _swait(slot):
        pltpu.make_async_copy(rows_v.at[slot], aggr.at[dst_v.at[slot]],
                              ssem.at[slot]).wait()

    for b in range(NBUF):
        _sdstart(b, b)
        _sdwait(b, b)
        _unpack(b)
        _gstart(b)

    def _pair(p, carry):
        for b in range(NBUF):
            j = p * NBUF + b

            @pl.when(j < NBATCH)
            def _():
                jn = jnp.minimum(j + NBUF, NBATCH - 1)
                _gwait(b)
                _sstart(b)

                @pl.when(j + NBUF < NBATCH)
                def _():
                    _sdstart(jn, b)
                _swait(b)

                @pl.when(j + NBUF < NBATCH)
                def _():
                    _sdwait(jn, b)
                    _unpack(b)
                    _gstart(b)
        return carry

    lax.fori_loop(0, (NBATCH + NBUF - 1) // NBUF, _pair, 0)
    plsc.subcore_barrier()

    obase = s * RPT

    def _wb(k, slot):
        pltpu.sync_copy(aggr.at[pl.ds(obase + k * B, B)], rows_v.at[slot])
        pltpu.make_async_copy(rows_v.at[slot],
                              out.at[c, pl.ds(obase + k * B, B)],
                              gsem.at[slot]).start()

    def _wbwait(k, slot):
        pltpu.make_async_copy(rows_v.at[slot],
                              out.at[c, pl.ds(obase + k * B, B)],
                              gsem.at[slot]).wait()

    for k in range(6):
        if k >= 2:
            _wbwait(k - 2, k & 1)
        _wb(k, k & 1)

    @pl.when(s < NS - 1)
    def _():
        _wbwait(4, 0)
        _wb(6, 0)
        _wbwait(5, 1)
        n_tail = RPT - 7 * B
        pltpu.sync_copy(aggr.at[pl.ds(obase + 7 * B, n_tail)],
                        rows_v.at[1, pl.ds(0, n_tail)])
        pltpu.sync_copy(rows_v.at[1, pl.ds(0, n_tail)],
                        out.at[c, pl.ds(obase + 7 * B, n_tail)])
        _wbwait(6, 0)

    @pl.when(s == NS - 1)
    def _():
        _wbwait(4, 0)
        _wbwait(5, 1)
        n_tail = RPT_LAST - 6 * B
        pltpu.sync_copy(aggr.at[pl.ds(obase + 6 * B, n_tail)],
                        rows_v.at[0, pl.ds(0, n_tail)])
        pltpu.sync_copy(rows_v.at[0, pl.ds(0, n_tail)],
                        out.at[c, pl.ds(obase + 6 * B, n_tail)])


_segsum_cache = []


def _segsum(h2, sd3):
    if not _segsum_cache:
        _segsum_cache.append(pl.kernel(
            _segsum_body,
            out_type=jax.ShapeDtypeStruct((NC, NAGG, DH), jnp.float32),
            mesh=plsc.VectorSubcoreMesh(core_axis_name="c",
                                        subcore_axis_name="s"),
            scratch_types=[
                pltpu.VMEM((NBUF, B), jnp.int32),
                pltpu.VMEM((NBUF, B), jnp.int32),
                pltpu.VMEM((NBUF, B), jnp.int32),
                pltpu.VMEM((NBUF, B, DH), jnp.float32),
                pltpu.VMEM_SHARED((NAGG, DH), jnp.float32),
                pltpu.SemaphoreType.DMA((NBUF,)),
                pltpu.SemaphoreType.DMA((NBUF,)),
                pltpu.SemaphoreType.DMA((NBUF,)),
            ],
            compiler_params=pltpu.CompilerParams(use_tc_tiling_on_sc=False),
        ))
    return _segsum_cache[0](h2, sd3)


def _emb_body(x_ref, tab_ref, out_ref):
    xb = x_ref[...]
    iota = lax.broadcasted_iota(jnp.int32, (1, VOC), 1)
    mh = (xb[:, 0:1] == iota).astype(jnp.float32)
    for i in range(1, 9):
        mh = mh + (xb[:, i:i + 1] == iota).astype(jnp.float32)
    out_ref[...] = jnp.dot(mh, tab_ref[...], preferred_element_type=jnp.float32)


def _emb(xoff, tabp):
    return pl.pallas_call(
        _emb_body,
        grid=(N // RBLK,),
        in_specs=[
            pl.BlockSpec((RBLK, 9), lambda i: (i, 0)),
            pl.BlockSpec((VOC, D), lambda i: (0, 0)),
        ],
        out_specs=pl.BlockSpec((RBLK, D), lambda i: (i, 0)),
        out_shape=jax.ShapeDtypeStruct((N, D), jnp.float32),
    )(xoff, tabp)


def _dense_body(h_ref, a_ref, er_ref, w1_ref, b1_ref, w2_ref, b2_ref,
                hout_ref, s_ref, q_ref):
    i = pl.program_id(0)
    a = a_ref[...]
    aggr = jnp.concatenate([a[0], a[1]], axis=1)
    z = er_ref[...] * h_ref[...] + aggr
    t = jnp.maximum(
        jnp.dot(z, w1_ref[...], preferred_element_type=jnp.float32)
        + b1_ref[...], 0.0)
    hu = (jnp.dot(t, w2_ref[...], preferred_element_type=jnp.float32)
          + b2_ref[...])
    hout_ref[...] = hu
    ps = jnp.sum(hu, axis=0, keepdims=True)
    pq = jnp.sum(hu * hu, axis=0, keepdims=True)

    @pl.when(i == 0)
    def _():
        s_ref[...] = ps
        q_ref[...] = pq

    @pl.when(i != 0)
    def _():
        s_ref[...] = s_ref[...] + ps
        q_ref[...] = q_ref[...] + pq


def _dense(h, aggr2, er, w1p, b1p, w2p, b2p):
    return pl.pallas_call(
        _dense_body,
        grid=(N // RBLK,),
        in_specs=[
            pl.BlockSpec((RBLK, D), lambda i: (i, 0)),
            pl.BlockSpec((NC, RBLK, DH), lambda i: (0, i, 0)),
            pl.BlockSpec((1, D), lambda i: (0, 0)),
            pl.BlockSpec((D, H), lambda i: (0, 0)),
            pl.BlockSpec((1, H), lambda i: (0, 0)),
            pl.BlockSpec((H, D), lambda i: (0, 0)),
            pl.BlockSpec((1, D), lambda i: (0, 0)),
        ],
        out_specs=[
            pl.BlockSpec((RBLK, D), lambda i: (i, 0)),
            pl.BlockSpec((1, D), lambda i: (0, 0)),
            pl.BlockSpec((1, D), lambda i: (0, 0)),
        ],
        out_shape=[
            jax.ShapeDtypeStruct((N, D), jnp.float32),
            jax.ShapeDtypeStruct((1, D), jnp.float32),
            jax.ShapeDtypeStruct((1, D), jnp.float32),
        ],
    )(h, aggr2, er, w1p, b1p, w2p, b2p)


def _bn_relu_body(h_ref, s_ref, q_ref, g_ref, b_ref, out_ref):
    mean = s_ref[...] * (1.0 / N)
    var = q_ref[...] * (1.0 / N) - mean * mean
    inv = g_ref[...] * lax.rsqrt(var + 1e-5)
    y = (h_ref[...] - mean) * inv + b_ref[...]
    out_ref[...] = jnp.maximum(y, 0.0)


def _bn_relu(hu, s, q, g, b):
    return pl.pallas_call(
        _bn_relu_body,
        grid=(N // RBLK,),
        in_specs=[
            pl.BlockSpec((RBLK, D), lambda i: (i, 0)),
            pl.BlockSpec((1, D), lambda i: (0, 0)),
            pl.BlockSpec((1, D), lambda i: (0, 0)),
            pl.BlockSpec((1, D), lambda i: (0, 0)),
            pl.BlockSpec((1, D), lambda i: (0, 0)),
        ],
        out_specs=pl.BlockSpec((RBLK, D), lambda i: (i, 0)),
        out_shape=jax.ShapeDtypeStruct((N, D), jnp.float32),
    )(hu, s, q, g, b)


def _dense_last_body(h_ref, a_ref, er_ref, w1_ref, b1_ref, w2_ref, b2_ref,
                     s_ref, q_ref):
    i = pl.program_id(0)
    a = a_ref[...]
    aggr = jnp.concatenate([a[0], a[1]], axis=1)
    z = er_ref[...] * h_ref[...] + aggr
    t = jnp.maximum(
        jnp.dot(z, w1_ref[...], preferred_element_type=jnp.float32)
        + b1_ref[...], 0.0)
    hu = (jnp.dot(t, w2_ref[...], preferred_element_type=jnp.float32)
          + b2_ref[...])
    ps = jnp.sum(hu, axis=0, keepdims=True)
    pq = jnp.sum(hu * hu, axis=0, keepdims=True)

    @pl.when(i == 0)
    def _():
        s_ref[...] = ps
        q_ref[...] = pq

    @pl.when(i != 0)
    def _():
        s_ref[...] = s_ref[...] + ps
        q_ref[...] = q_ref[...] + pq


def _dense_last(h, aggr2, er, w1p, b1p, w2p, b2p):
    return pl.pallas_call(
        _dense_last_body,
        grid=(N // RBLK,),
        in_specs=[
            pl.BlockSpec((RBLK, D), lambda i: (i, 0)),
            pl.BlockSpec((NC, RBLK, DH), lambda i: (0, i, 0)),
            pl.BlockSpec((1, D), lambda i: (0, 0)),
            pl.BlockSpec((D, H), lambda i: (0, 0)),
            pl.BlockSpec((1, H), lambda i: (0, 0)),
            pl.BlockSpec((H, D), lambda i: (0, 0)),
            pl.BlockSpec((1, D), lambda i: (0, 0)),
        ],
        out_specs=[
            pl.BlockSpec((1, D), lambda i: (0, 0)),
            pl.BlockSpec((1, D), lambda i: (0, 0)),
        ],
        out_shape=[
            jax.ShapeDtypeStruct((1, D), jnp.float32),
            jax.ShapeDtypeStruct((1, D), jnp.float32),
        ],
    )(h, aggr2, er, w1p, b1p, w2p, b2p)


def _softplus(x):
    return jnp.maximum(x, 0.0) + jnp.log1p(jnp.exp(-jnp.abs(x)))


def _head_body(s_ref, q_ref, g4_ref, b4_ref, fw_ref, fb_ref, w0_ref, b0_ref,
               w1_ref, b1_ref, w2_ref, b2_ref, out_ref):
    mean = s_ref[...] * (1.0 / N)
    var = q_ref[...] * (1.0 / N) - mean * mean
    inv = g4_ref[...] * lax.rsqrt(var + 1e-5)
    g = (s_ref[...] * (1.0 / N) - mean) * inv + b4_ref[...]
    f = (jnp.dot(g, fw_ref[...], preferred_element_type=jnp.float32)
         + fb_ref[...])
    f = _softplus(jnp.dot(f, w0_ref[...], preferred_element_type=jnp.float32)
                  + b0_ref[...])
    f = _softplus(jnp.dot(f, w1_ref[...], preferred_element_type=jnp.float32)
                  + b1_ref[...])
    out_ref[...] = (jnp.dot(f, w2_ref[...], preferred_element_type=jnp.float32)
                    + b2_ref[...])


def _head(s, q, g4, b4, fw, fb, w0, b0, w1, b1, w2, b2):
    return pl.pallas_call(
        _head_body,
        out_shape=jax.ShapeDtypeStruct((1, 128), jnp.float32),
    )(s, q, g4, b4, fw, fb, w0, b0, w1, b1, w2, b2)


def kernel(x, edge_index, params):
    offs = jnp.asarray(_OFFS, jnp.int32)
    xoff = x.astype(jnp.int32) + offs[None, :]
    tabp = _pad2(jnp.concatenate(params["emb"], axis=0), VOC, D)

    src = edge_index[0].astype(jnp.int32)
    dst = edge_index[1].astype(jnp.int32)
    sd = (src << 14) | dst
    sd3 = sd.reshape(NS, NBATCH, B)

    h = _emb(xoff, tabp)
    for l in range(NLAYER):
        p = params["gnn"][l]
        aggr2 = _segsum(h.reshape(2 * N, DH), sd3)
        er = jnp.broadcast_to(1.0 + p["eps"], (1, D))
        w1p = _pad2(p["W1"], D, H)
        b1p = _padrow(p["b1"], H)
        w2p = _pad2(p["W2"], H, D)
        b2p = _padrow(p["b2"], D)
        gp = _padrow(p["bn_g"], D)
        bp = _padrow(p["bn_b"], D)
        if l < NLAYER - 1:
            hu, s, q = _dense(h, aggr2, er, w1p, b1p, w2p, b2p)
            h = _bn_relu(hu, s, q, gp, bp)
        else:
            s4, q4 = _dense_last(h, aggr2, er, w1p, b1p, w2p, b2p)
            g4, b4 = gp, bp

    fw = _pad2(params["feat_W"], D, FEAT)
    fb = _padrow(params["feat_b"], FEAT)
    w0 = params["p0W"]
    b0 = _padrow(params["p0b"], FEAT // 2)
    w1 = params["p1W"]
    b1 = _padrow(params["p1b"], FEAT // 2)
    w2 = _pad2(params["p2W"], FEAT // 2, 128)
    b2 = _padrow(params["p2b"], 128)
    out = _head(s4, q4, g4, b4, fw, fb, w0, b0, w1, b1, w2, b2)
    return out[:, :2]
